# baseline (speedup 1.0000x reference)
import functools

import jax
import jax.numpy as jnp
from jax import lax
from jax.experimental import pallas as pl
from jax.experimental.pallas import tpu as pltpu
from jax.experimental.pallas import tpu_sc as plsc

INF = 1000000000.0

NC = 2
NS = 16
NW = NC * NS
L = 16

N_X = 16777216
N_DATA = 4194304
M = N_DATA + 1
GAP = 64
N_COARSE = N_DATA // GAP + 1
N_COARSE_PAD = 65544

B = 2048
PER_W = N_X // NW
N_BATCH = PER_W // B
CHUNKS = B // L
SUB = B // 128
ROWS = N_DATA // L


def _cdf_body(x_hbm, pad_hbm, rows_hbm, coarse_hbm, params_hbm, out_hbm,
              coarse_v, xc_v, b0_v, lov_v, hiv_v, idx_v, g_v, rows_v, out_v,
              par_v, sem):
    wid = lax.axis_index("s") * NC + lax.axis_index("c")
    pltpu.sync_copy(coarse_hbm, coarse_v)
    pltpu.sync_copy(params_hbm, par_v)
    dmin = par_v[pl.ds(0, L)]
    dmax = par_v[pl.ds(L, L)]
    lane = lax.iota(jnp.int32, L)
    base0 = wid * PER_W

    def fire_words(dst_ref):
        cps = [
            pltpu.async_copy(
                pad_hbm.at[idx_v.at[pl.ds(j * 128, 128)]],
                dst_ref.at[pl.ds(j * 128, 128)], sem)
            for j in range(SUB)
        ]
        for cp in cps:
            cp.wait()

    def batch_body(b, _):
        base = base0 + b * B
        pltpu.sync_copy(x_hbm.at[pl.ds(base, B)], xc_v)

        @plsc.parallel_loop(0, CHUNKS, unroll=8)
        def _coarse(i):
            sl = pl.ds(i * L, L)
            xcl = jnp.minimum(jnp.maximum(xc_v[sl], dmin), dmax)
            bidx = jnp.zeros((L,), jnp.int32)
            for r in range(16):
                cand = bidx + (32768 >> r)
                v = plsc.load_gather(coarse_v, [cand])
                bidx = jnp.where(v < xcl, cand, bidx)
            lov = plsc.load_gather(coarse_v, [bidx])
            hiv = plsc.load_gather(coarse_v, [bidx + 1])
            b0 = bidx * GAP
            xc_v[sl] = xcl
            b0_v[sl] = b0
            lov_v[sl] = lov
            hiv_v[sl] = hiv
            idx_v[sl] = b0 + 32

        with jax.named_scope("ph2_dma32"):
            fire_words(g_v)

        @plsc.parallel_loop(0, CHUNKS, unroll=8)
        def _upd32(i):
            sl = pl.ds(i * L, L)
            g = g_v[sl]
            cond = g < xc_v[sl]
            b0 = jnp.where(cond, b0_v[sl] + 32, b0_v[sl])
            b0_v[sl] = b0
            lov_v[sl] = jnp.where(cond, g, lov_v[sl])
            hiv_v[sl] = jnp.where(cond, hiv_v[sl], g)
            idx_v[sl] = b0 + 16

        with jax.named_scope("ph3_dma16"):
            fire_words(g_v)

        @plsc.parallel_loop(0, CHUNKS, unroll=8)
        def _upd16(i):
            sl = pl.ds(i * L, L)
            g = g_v[sl]
            cond = g < xc_v[sl]
            b0 = jnp.where(cond, b0_v[sl] + 16, b0_v[sl])
            b0_v[sl] = b0
            lov_v[sl] = jnp.where(cond, g, lov_v[sl])
            hiv_v[sl] = jnp.where(cond, hiv_v[sl], g)
            idx_v[sl] = lax.shift_right_logical(b0, 4)

        with jax.named_scope("ph4_rowdma"):
            cps = [
                pltpu.async_copy(
                    rows_hbm.at[idx_v.at[pl.ds(j * 128, 128)]],
                    rows_v.at[pl.ds(j * 128, 128)], sem)
                for j in range(SUB)
            ]
            for cp in cps:
                cp.wait()

        @plsc.parallel_loop(0, CHUNKS, unroll=8)
        def _local(i):
            sl = pl.ds(i * L, L)
            q_idx = i * L + lane
            xcl = xc_v[sl]
            lov = lov_v[sl]
            hiv = hiv_v[sl]
            o = jnp.zeros((L,), jnp.int32)
            for w in (8, 4, 2, 1):
                cand = o + w
                v = plsc.load_gather(rows_v, [q_idx, cand])
                cond = v < xcl
                o = jnp.where(cond, cand, o)
                lov = jnp.where(cond, v, lov)
                hiv = jnp.where(cond, hiv, v)
            b0f = (b0_v[sl] + o).astype(jnp.float32)
            delta = (hiv - xcl) / (hiv - lov)
            out_v[sl] = (b0f - delta) * (1.0 / (M - 2))

        pltpu.sync_copy(out_v, out_hbm.at[pl.ds(base, B)])
        return 0

    with jax.named_scope("batches"):
        lax.fori_loop(0, N_BATCH, batch_body, 0)


@jax.jit
def kernel(x, data):
    sorted_data = jnp.sort(data)
    pad = jnp.concatenate(
        [jnp.full((1,), -INF, dtype=jnp.float32), sorted_data])
    rows = pad[:N_DATA].reshape(ROWS, L)
    coarse = pad[::GAP]
    coarse = jnp.concatenate(
        [coarse, jnp.full((N_COARSE_PAD - N_COARSE,), INF, jnp.float32)])
    params = jnp.concatenate([
        jnp.broadcast_to(sorted_data[0], (L,)),
        jnp.broadcast_to(sorted_data[-1], (L,)),
    ]).astype(jnp.float32)

    mesh = plsc.VectorSubcoreMesh(core_axis_name="c", subcore_axis_name="s")
    run = functools.partial(
        pl.kernel,
        mesh=mesh,
        compiler_params=pltpu.CompilerParams(
            needs_layout_passes=False, use_tc_tiling_on_sc=False),
        out_type=jax.ShapeDtypeStruct((N_X,), jnp.float32),
        scratch_types=[
            pltpu.VMEM((N_COARSE_PAD,), jnp.float32),
            pltpu.VMEM((B,), jnp.float32),
            pltpu.VMEM((B,), jnp.int32),
            pltpu.VMEM((B,), jnp.float32),
            pltpu.VMEM((B,), jnp.float32),
            pltpu.VMEM((B,), jnp.int32),
            pltpu.VMEM((B,), jnp.float32),
            pltpu.VMEM((B, L), jnp.float32),
            pltpu.VMEM((B,), jnp.float32),
            pltpu.VMEM((2 * L,), jnp.float32),
            pltpu.SemaphoreType.DMA,
        ],
    )(_cdf_body)
    return run(x, pad, rows, coarse, params)


# speedup vs baseline: 1.2337x; 1.2337x over previous
import functools

import jax
import jax.numpy as jnp
from jax import lax
from jax.experimental import pallas as pl
from jax.experimental.pallas import tpu as pltpu
from jax.experimental.pallas import tpu_sc as plsc

INF = 1000000000.0

NC = 2
NS = 16
NW = NC * NS
L = 16

N_X = 16777216
N_DATA = 4194304
M = N_DATA + 1
GAP = 64
N_COARSE = N_DATA // GAP + 1
N_COARSE_PAD = 65544

B = 2048
PER_W = N_X // NW
N_BATCH = PER_W // B
CHUNKS = B // L
SUB = B // 128
ROWS = N_DATA // L

SEGS = ((0, 44), (44, 88), (88, 128))


def _cdf_body(x_hbm, pad_hbm, rows_hbm, coarse_hbm, params_hbm, out_hbm,
              coarse_v,
              xc_a, b0_a, lov_a, hiv_a, idx_a,
              xc_b, b0_b, lov_b, hiv_b, idx_b,
              g_v, rows_v, out_v, par_v, sem):
    wid = lax.axis_index("s") * NC + lax.axis_index("c")
    pltpu.sync_copy(coarse_hbm, coarse_v)
    pltpu.sync_copy(params_hbm, par_v)
    dmin = par_v[pl.ds(0, L)]
    dmax = par_v[pl.ds(L, L)]
    lane = lax.iota(jnp.int32, L)
    base0 = wid * PER_W

    set_a = (xc_a, b0_a, lov_a, hiv_a, idx_a)
    set_b = (xc_b, b0_b, lov_b, hiv_b, idx_b)

    def coarse_seg(xc_v, b0_v, lov_v, hiv_v, idx_v, lo, hi):
        @plsc.parallel_loop(lo, hi, unroll=4)
        def _c(i):
            sl = pl.ds(i * L, L)
            xcl = jnp.minimum(jnp.maximum(xc_v[sl], dmin), dmax)
            bidx = jnp.zeros((L,), jnp.int32)
            for r in range(16):
                cand = bidx + (32768 >> r)
                v = plsc.load_gather(coarse_v, [cand])
                bidx = jnp.where(v < xcl, cand, bidx)
            lov = plsc.load_gather(coarse_v, [bidx])
            hiv = plsc.load_gather(coarse_v, [bidx + 1])
            b0 = bidx * GAP
            xc_v[sl] = xcl
            b0_v[sl] = b0
            lov_v[sl] = lov
            hiv_v[sl] = hiv
            idx_v[sl] = b0 + 32

    def fire_words(idx_v):
        def fj(j, _):
            sl = pl.ds(j * 128, 128)
            pltpu.async_copy(pad_hbm.at[idx_v.at[sl]], g_v.at[sl], sem)
            return 0
        lax.fori_loop(0, SUB, fj, 0)

    def drain_words():
        pltpu.make_async_copy(pad_hbm.at[pl.ds(0, B)], g_v, sem).wait()

    def fire_rows(idx_v):
        def fj(j, _):
            sl = pl.ds(j * 128, 128)
            pltpu.async_copy(rows_hbm.at[idx_v.at[sl]], rows_v.at[sl], sem)
            return 0
        lax.fori_loop(0, SUB, fj, 0)

    def drain_rows():
        pltpu.make_async_copy(rows_hbm.at[pl.ds(0, B)], rows_v, sem).wait()

    def upd(xc_v, b0_v, lov_v, hiv_v, idx_v, w, emit_row):
        @plsc.parallel_loop(0, CHUNKS, unroll=8)
        def _u(i):
            sl = pl.ds(i * L, L)
            g = g_v[sl]
            cond = g < xc_v[sl]
            b0 = jnp.where(cond, b0_v[sl] + w, b0_v[sl])
            b0_v[sl] = b0
            lov_v[sl] = jnp.where(cond, g, lov_v[sl])
            hiv_v[sl] = jnp.where(cond, hiv_v[sl], g)
            if emit_row:
                idx_v[sl] = lax.shift_right_logical(b0, 4)
            else:
                idx_v[sl] = b0 + 16

    def local_interp(xc_v, b0_v, lov_v, hiv_v, idx_v):
        @plsc.parallel_loop(0, CHUNKS, unroll=4)
        def _l(i):
            sl = pl.ds(i * L, L)
            q_idx = i * L + lane
            xcl = xc_v[sl]
            lov = lov_v[sl]
            hiv = hiv_v[sl]
            o = jnp.zeros((L,), jnp.int32)
            for w in (8, 4, 2, 1):
                cand = o + w
                v = plsc.load_gather(rows_v, [q_idx, cand])
                cond = v < xcl
                o = jnp.where(cond, cand, o)
                lov = jnp.where(cond, v, lov)
                hiv = jnp.where(cond, hiv, v)
            b0f = (b0_v[sl] + o).astype(jnp.float32)
            delta = (hiv - xcl) / (hiv - lov)
            out_v[sl] = (b0f - delta) * (1.0 / (M - 2))

    def fine_half(mine, other, batch_n):
        has_next = batch_n + 1 < N_BATCH

        fire_words(mine[4])

        @pl.when(has_next)
        def _s0():
            coarse_seg(*other, *SEGS[0])

        drain_words()
        upd(*mine, 32, False)
        fire_words(mine[4])

        @pl.when(has_next)
        def _s1():
            coarse_seg(*other, *SEGS[1])

        drain_words()
        upd(*mine, 16, True)
        fire_rows(mine[4])

        @pl.when(has_next)
        def _s2():
            coarse_seg(*other, *SEGS[2])

        drain_rows()
        local_interp(*mine)
        pltpu.sync_copy(out_v, out_hbm.at[pl.ds(base0 + batch_n * B, B)])

    def load_x(batch_n, xc_v):
        @pl.when(batch_n < N_BATCH)
        def _():
            pltpu.sync_copy(x_hbm.at[pl.ds(base0 + batch_n * B, B)], xc_v)

    pltpu.sync_copy(x_hbm.at[pl.ds(base0, B)], xc_a)
    coarse_seg(*set_a, 0, CHUNKS)
    pltpu.sync_copy(x_hbm.at[pl.ds(base0 + B, B)], xc_b)

    def pair_body(k, _):
        n_a = 2 * k
        fine_half(set_a, set_b, n_a)
        load_x(n_a + 2, xc_a)
        fine_half(set_b, set_a, n_a + 1)
        load_x(n_a + 3, xc_b)
        return 0

    lax.fori_loop(0, N_BATCH // 2, pair_body, 0)


@jax.jit
def kernel(x, data):
    sorted_data = jnp.sort(data)
    pad = jnp.concatenate(
        [jnp.full((1,), -INF, dtype=jnp.float32), sorted_data])
    rows = pad[:N_DATA].reshape(ROWS, L)
    coarse = pad[::GAP]
    coarse = jnp.concatenate(
        [coarse, jnp.full((N_COARSE_PAD - N_COARSE,), INF, jnp.float32)])
    params = jnp.concatenate([
        jnp.broadcast_to(sorted_data[0], (L,)),
        jnp.broadcast_to(sorted_data[-1], (L,)),
    ]).astype(jnp.float32)

    mesh = plsc.VectorSubcoreMesh(core_axis_name="c", subcore_axis_name="s")
    run = functools.partial(
        pl.kernel,
        mesh=mesh,
        compiler_params=pltpu.CompilerParams(
            needs_layout_passes=False, use_tc_tiling_on_sc=False),
        out_type=jax.ShapeDtypeStruct((N_X,), jnp.float32),
        scratch_types=[
            pltpu.VMEM((N_COARSE_PAD,), jnp.float32),
            pltpu.VMEM((B,), jnp.float32),
            pltpu.VMEM((B,), jnp.int32),
            pltpu.VMEM((B,), jnp.float32),
            pltpu.VMEM((B,), jnp.float32),
            pltpu.VMEM((B,), jnp.int32),
            pltpu.VMEM((B,), jnp.float32),
            pltpu.VMEM((B,), jnp.int32),
            pltpu.VMEM((B,), jnp.float32),
            pltpu.VMEM((B,), jnp.float32),
            pltpu.VMEM((B,), jnp.int32),
            pltpu.VMEM((B,), jnp.float32),
            pltpu.VMEM((B, L), jnp.float32),
            pltpu.VMEM((B,), jnp.float32),
            pltpu.VMEM((2 * L,), jnp.float32),
            pltpu.SemaphoreType.DMA,
        ],
    )(_cdf_body)
    return run(x, pad, rows, coarse, params)
